# baseline (speedup 1.0000x reference)
import functools

import jax
import jax.numpy as jnp
import numpy as np
from jax import lax
from jax.experimental import pallas as pl
from jax.experimental.pallas import tpu as pltpu
from jax.experimental.pallas import tpu_sc as plsc

_B = 4096
_D = 1024
_N_TAKE = 40
_NW = 16
_SLOTS = 3
_STRIDE = 8
_LANES = 16
_CHUNKS = _D // _LANES

_flat = np.array([
    3297861, 16046192, 297537, 10150400, 16162907, 7472413, 5652315,
    13627135, 2373379, 6843762, 659676, 3043796, 594201, 1224974, 5210207,
    4573914, 7984611, 14510481, 7729220, 5436461, 5724811, 407871, 9526120,
    823845, 9283720, 9750771, 1852445, 6763374, 6179668, 6466523, 14830220,
    16017240, 3560071, 10157330, 14827734, 16738087, 10017649, 7189393,
    10103154, 16370316,
], dtype=np.int64)
_i_all = (_flat // _B).astype(np.int32)
_j_all = (_flat % _B).astype(np.int32)

_i_arr = np.zeros((_NW * _STRIDE,), np.int32)
_j_arr = np.zeros((_NW * _STRIDE,), np.int32)
for _s in range(_SLOTS):
    for _w in range(_NW):
        _p = _s * _NW + _w
        if _p < _N_TAKE:
            _i_arr[_w * _STRIDE + _s] = _i_all[_p]
            _j_arr[_w * _STRIDE + _s] = _j_all[_p]

_mesh = plsc.VectorSubcoreMesh(
    core_axis_name="c", subcore_axis_name="s", num_cores=1, num_subcores=_NW
)


def _newton_sqrt_vec(x):
    m = x
    s = jnp.full((_LANES,), 1.0, jnp.float32)
    for t in (32, 16, 8, 4, 2, 1):
        big = m >= 4.0 ** t
        m = jnp.where(big, m * 4.0 ** -t, m)
        s = jnp.where(big, s * 2.0 ** t, s)
    for t in (32, 32, 16, 8, 4, 2, 1):
        small = m < 4.0 ** (1 - t)
        m = jnp.where(small, m * 4.0 ** t, m)
        s = jnp.where(small, s * 2.0 ** -t, s)
    y = 0.59 + 0.4245 * m
    for _ in range(4):
        y = 0.5 * (y + m / y)
    return jnp.where(x > 0.0, s * y, jnp.zeros((_LANES,), jnp.float32))


_SCRATCH_TYPES = [
    pltpu.VMEM((_STRIDE,), jnp.int32),
    pltpu.VMEM((_STRIDE,), jnp.int32),
    pltpu.VMEM((_STRIDE, _D), jnp.float32),
    pltpu.VMEM((_STRIDE, _D), jnp.float32),
    pltpu.VMEM((_LANES,), jnp.float32),
    pltpu.VMEM((_NW, _LANES), jnp.float32),
    pltpu.SemaphoreType.DMA,
    pltpu.SemaphoreType.DMA,
]


def _sc_loss_body(batch_hbm, output_hbm, i_hbm, j_hbm, out_hbm,
                  iv, jv, arows, brows, acc, gath, sem_a, sem_b):
    wid = lax.axis_index("s")
    base = wid * _STRIDE
    pltpu.sync_copy(i_hbm.at[pl.ds(base, _STRIDE)], iv)
    pltpu.sync_copy(j_hbm.at[pl.ds(base, _STRIDE)], jv)
    cpa = pltpu.async_copy(batch_hbm.at[iv], arows, sem_a)
    cpb = pltpu.async_copy(output_hbm.at[jv], brows, sem_b)
    cpa.wait()
    cpb.wait()
    part = jnp.zeros((_LANES,), jnp.float32)
    for s in range(_SLOTS - 1):
        for t in range(_CHUNKS):
            d = (arows[s, pl.ds(t * _LANES, _LANES)]
                 - brows[s, pl.ds(t * _LANES, _LANES)])
            part = part + d * d
    acc[...] = part
    @pl.when(wid < _N_TAKE - 2 * _NW)
    def _():
        p2 = jnp.zeros((_LANES,), jnp.float32)
        s = _SLOTS - 1
        for t in range(_CHUNKS):
            d = (arows[s, pl.ds(t * _LANES, _LANES)]
                 - brows[s, pl.ds(t * _LANES, _LANES)])
            p2 = p2 + d * d
        acc[...] = acc[...] + p2

    pltpu.sync_copy(acc, out_hbm.at[wid])
    plsc.subcore_barrier()

    pltpu.sync_copy(out_hbm, gath)
    tot = gath[0, :]
    for w in range(1, _NW):
        tot = tot + gath[w, :]
    lanes = lax.iota(jnp.int32, _LANES)
    for k in (8, 4, 2, 1):
        tot = tot + tot.at[lanes ^ k].get(mode="promise_in_bounds")
    acc[...] = _newton_sqrt_vec(tot)

    @pl.when(wid == 0)
    def _():
        pltpu.sync_copy(acc, out_hbm.at[0])


_sc_loss = pl.kernel(
    _sc_loss_body,
    out_type=jax.ShapeDtypeStruct((_NW, _LANES), jnp.float32),
    mesh=_mesh,
    scratch_types=_SCRATCH_TYPES,
)


def kernel(batch, output, optimization_mode=0):
    i_const = jnp.asarray(_i_arr)
    j_const = jnp.asarray(_j_arr)
    out = _sc_loss(batch, output, i_const, j_const)
    return out[0, 0]


# speedup vs baseline: 1.2244x; 1.2244x over previous
import jax
import jax.numpy as jnp
import numpy as np
from jax import lax
from jax.experimental import pallas as pl
from jax.experimental.pallas import tpu as pltpu
from jax.experimental.pallas import tpu_sc as plsc

_B = 4096
_D = 1024
_N_TAKE = 40
_NW = 16
_SLOTS = 3
_GLEN = 4
_STRIDE = 8
_LANES = 16
_CHUNKS = _D // _LANES

_flat = np.array([
    3297861, 16046192, 297537, 10150400, 16162907, 7472413, 5652315,
    13627135, 2373379, 6843762, 659676, 3043796, 594201, 1224974, 5210207,
    4573914, 7984611, 14510481, 7729220, 5436461, 5724811, 407871, 9526120,
    823845, 9283720, 9750771, 1852445, 6763374, 6179668, 6466523, 14830220,
    16017240, 3560071, 10157330, 14827734, 16738087, 10017649, 7189393,
    10103154, 16370316,
], dtype=np.int64)
_i_all = (_flat // _B).astype(np.int32)
_j_all = (_flat % _B).astype(np.int32)

_i_arr = np.zeros((_NW * _STRIDE,), np.int32)
_j_arr = np.zeros((_NW * _STRIDE,), np.int32)
for _s in range(_SLOTS):
    for _w in range(_NW):
        _p = _s * _NW + _w
        if _p < _N_TAKE:
            _i_arr[_w * _STRIDE + _s] = _i_all[_p]
            _j_arr[_w * _STRIDE + _s] = _j_all[_p]

_mesh = plsc.VectorSubcoreMesh(
    core_axis_name="c", subcore_axis_name="s", num_cores=1, num_subcores=_NW
)


def _newton_sqrt_vec(x):
    m = x
    s = jnp.full((_LANES,), 1.0, jnp.float32)
    for t in (32, 16, 8, 4, 2, 1):
        big = m >= 4.0 ** t
        m = jnp.where(big, m * 4.0 ** -t, m)
        s = jnp.where(big, s * 2.0 ** t, s)
    for t in (32, 32, 16, 8, 4, 2, 1):
        small = m < 4.0 ** (1 - t)
        m = jnp.where(small, m * 4.0 ** t, m)
        s = jnp.where(small, s * 2.0 ** -t, s)
    y = 0.59 + 0.4245 * m
    for _ in range(4):
        y = 0.5 * (y + m / y)
    return jnp.where(x > 0.0, s * y, jnp.zeros((_LANES,), jnp.float32))


_SCRATCH_TYPES = [
    pltpu.VMEM((_STRIDE,), jnp.int32),
    pltpu.VMEM((_STRIDE,), jnp.int32),
    pltpu.VMEM((_GLEN, _D), jnp.float32),
    pltpu.VMEM((_GLEN, _D), jnp.float32),
    pltpu.VMEM((_LANES,), jnp.float32),
    pltpu.VMEM((_NW, _LANES), jnp.float32),
    pltpu.SemaphoreType.DMA,
    pltpu.SemaphoreType.DMA,
]


def _sc_loss_body(batch_hbm, output_hbm, i_hbm, j_hbm, out_hbm,
                  iv, jv, arows, brows, acc, gath, sem_a, sem_b):
    wid = lax.axis_index("s")
    base = wid * _STRIDE
    cpi = pltpu.async_copy(i_hbm.at[pl.ds(base, _STRIDE)], iv, sem_a)
    cpj = pltpu.async_copy(j_hbm.at[pl.ds(base, _STRIDE)], jv, sem_b)
    cpi.wait()
    cpj.wait()
    cpa = pltpu.async_copy(batch_hbm.at[iv.at[pl.ds(0, _GLEN)]], arows, sem_a)
    cpb = pltpu.async_copy(output_hbm.at[jv.at[pl.ds(0, _GLEN)]], brows, sem_b)
    cpa.wait()
    cpb.wait()
    part = jnp.zeros((_LANES,), jnp.float32)
    for s in range(_SLOTS - 1):
        for t in range(_CHUNKS):
            d = (arows[s, pl.ds(t * _LANES, _LANES)]
                 - brows[s, pl.ds(t * _LANES, _LANES)])
            part = part + d * d
    acc[...] = part
    @pl.when(wid < _N_TAKE - 2 * _NW)
    def _():
        p2 = jnp.zeros((_LANES,), jnp.float32)
        s = _SLOTS - 1
        for t in range(_CHUNKS):
            d = (arows[s, pl.ds(t * _LANES, _LANES)]
                 - brows[s, pl.ds(t * _LANES, _LANES)])
            p2 = p2 + d * d
        acc[...] = acc[...] + p2

    pltpu.sync_copy(acc, out_hbm.at[wid])
    plsc.subcore_barrier()

    @pl.when(wid == 0)
    def _():
        pltpu.sync_copy(out_hbm, gath)

    tot = gath[0, :]
    for w in range(1, _NW):
        tot = tot + gath[w, :]
    lanes = lax.iota(jnp.int32, _LANES)
    for k in (8, 4, 2, 1):
        tot = tot + tot.at[lanes ^ k].get(mode="promise_in_bounds")
    acc[...] = _newton_sqrt_vec(tot)

    @pl.when(wid == 0)
    def _():
        pltpu.sync_copy(acc, out_hbm.at[0])


_sc_loss = pl.kernel(
    _sc_loss_body,
    out_type=jax.ShapeDtypeStruct((_NW, _LANES), jnp.float32),
    mesh=_mesh,
    scratch_types=_SCRATCH_TYPES,
)


def kernel(batch, output, optimization_mode=0):
    i_const = jnp.asarray(_i_arr)
    j_const = jnp.asarray(_j_arr)
    out = _sc_loss(batch, output, i_const, j_const)
    return out[0, 0]
